# baseline (speedup 1.0000x reference)
import functools

import jax
import jax.numpy as jnp
from jax import lax
from jax.experimental import pallas as pl
from jax.experimental.pallas import tpu as pltpu
from jax.experimental.pallas import tpu_sc as plsc

_NC = 2
_NS = 16
_NW = _NC * _NS
_CH = 128


def _make_lookup(B, V, D, b_per_w, n_ch):
    mesh = plsc.VectorSubcoreMesh(core_axis_name="c", subcore_axis_name="s")

    @functools.partial(
        pl.kernel,
        mesh=mesh,
        out_type=jax.ShapeDtypeStruct((B, D), jnp.float32),
        scratch_types=[
            pltpu.VMEM((b_per_w,), jnp.int32),
            pltpu.VMEM((b_per_w, D), jnp.float32),
            pltpu.SemaphoreType.DMA,
            pltpu.SemaphoreType.DMA,
        ],
    )
    def lookup(table_hbm, idx_hbm, out_hbm, idx_v, rows_v, sem, isem):
        wid = lax.axis_index("s") * _NC + lax.axis_index("c")
        base = wid * b_per_w
        idx_copies = [
            pltpu.async_copy(
                idx_hbm.at[pl.ds(base + j * _CH, _CH)],
                idx_v.at[pl.ds(j * _CH, _CH)],
                isem,
            )
            for j in range(n_ch)
        ]
        gathers = []
        for j in range(n_ch):
            idx_copies[j].wait()
            gathers.append(
                pltpu.async_copy(
                    table_hbm.at[idx_v.at[pl.ds(j * _CH, _CH)]],
                    rows_v.at[pl.ds(j * _CH, _CH)],
                    sem,
                )
            )
        outs = []
        for j in range(n_ch):
            gathers[j].wait()
            outs.append(
                pltpu.async_copy(
                    rows_v.at[pl.ds(j * _CH, _CH)],
                    out_hbm.at[pl.ds(base + j * _CH, _CH)],
                    isem,
                )
            )
        for c in outs:
            c.wait()

    return lookup


def kernel(nodes, table):
    (B,) = nodes.shape
    V, D = table.shape
    b_per_w = B // _NW
    n_ch = b_per_w // _CH
    return _make_lookup(B, V, D, b_per_w, n_ch)(table, nodes.astype(jnp.int32))


# speedup vs baseline: 1.0237x; 1.0237x over previous
import functools

import jax
import jax.numpy as jnp
from jax import lax
from jax.experimental import pallas as pl
from jax.experimental.pallas import tpu as pltpu
from jax.experimental.pallas import tpu_sc as plsc

_NC = 2
_NS = 16
_NW = _NC * _NS
_CH = 128


def _make_lookup(B, V, D, b_per_w, n_ch):
    mesh = plsc.VectorSubcoreMesh(core_axis_name="c", subcore_axis_name="s")

    @functools.partial(
        pl.kernel,
        mesh=mesh,
        out_type=jax.ShapeDtypeStruct((B, D), jnp.float32),
        scratch_types=[
            pltpu.VMEM((b_per_w,), jnp.int32),
            pltpu.VMEM((b_per_w, D), jnp.float32),
            pltpu.SemaphoreType.DMA,
            pltpu.SemaphoreType.DMA,
        ],
    )
    def lookup(table_hbm, idx_hbm, out_hbm, idx_v, rows_v, sem, isem):
        wid = lax.axis_index("s") * _NC + lax.axis_index("c")
        base = wid * b_per_w
        idx_copies = [
            pltpu.async_copy(
                idx_hbm.at[pl.ds(base + j * _CH, _CH)],
                idx_v.at[pl.ds(j * _CH, _CH)],
                isem,
            )
            for j in range(n_ch)
        ]
        gathers = []
        for j in range(n_ch):
            idx_copies[j].wait()
            gathers.append(
                pltpu.async_copy(
                    table_hbm.at[idx_v.at[pl.ds(j * _CH, _CH)]],
                    rows_v.at[pl.ds(j * _CH, _CH)],
                    sem,
                )
            )
        for c in gathers:
            c.wait()
        pltpu.sync_copy(rows_v, out_hbm.at[pl.ds(base, b_per_w)])

    return lookup


def kernel(nodes, table):
    (B,) = nodes.shape
    V, D = table.shape
    b_per_w = B // _NW
    n_ch = b_per_w // _CH
    return _make_lookup(B, V, D, b_per_w, n_ch)(table, nodes.astype(jnp.int32))
